# Initial kernel scaffold; baseline (speedup 1.0000x reference)
import jax
import jax.numpy as jnp
from jax.experimental import pallas as pl


def kernel(x, edge_index, batch, W1, as1, ad1, b1, g1, be1, W2, as2, ad2, b2, g2, be2, W3, as3, ad3, b3, g3, be3, lnW, lnb, l0W, l0b, l1W, l1b):
    raise NotImplementedError("write your pallas kernel here")


# speedup vs baseline: 1.0049x; 1.0049x over previous
import functools
import jax
import jax.numpy as jnp
from jax.experimental import pallas as pl
from jax.experimental.pallas import tpu as pltpu

_H = 8
_C = 128
_BLK = 1024


def _proj_body(x_ref, w_ref, asf_ref, adf_ref, eh_ref, xh_ref, al_ref):
    xh = jnp.dot(x_ref[...], w_ref[...], preferred_element_type=jnp.float32)
    xh_ref[...] = xh
    eh = eh_ref[...]
    als = jnp.dot(xh * asf_ref[...], eh, preferred_element_type=jnp.float32)
    ald = jnp.dot(xh * adf_ref[...], eh, preferred_element_type=jnp.float32)
    al_ref[...] = jnp.concatenate([als, ald], axis=1)


def _project(xp, W, a_s, a_d, eh):
    np_, din = xp.shape
    hc = W.shape[1]
    grid = (np_ // _BLK,)
    xh, al = pl.pallas_call(
        _proj_body,
        grid=grid,
        in_specs=[
            pl.BlockSpec((_BLK, din), lambda i: (i, 0)),
            pl.BlockSpec((din, hc), lambda i: (0, 0)),
            pl.BlockSpec((1, hc), lambda i: (0, 0)),
            pl.BlockSpec((1, hc), lambda i: (0, 0)),
            pl.BlockSpec((hc, _H), lambda i: (0, 0)),
        ],
        out_specs=[
            pl.BlockSpec((_BLK, hc), lambda i: (i, 0)),
            pl.BlockSpec((_BLK, 2 * _H), lambda i: (i, 0)),
        ],
        out_shape=[
            jax.ShapeDtypeStruct((np_, hc), jnp.float32),
            jax.ShapeDtypeStruct((np_, 2 * _H), jnp.float32),
        ],
    )(xp, W, a_s, a_d, eh)
    return xh, al


def _gat_layer(xp, n, src, dst, W, a_s, a_d, b, eh):
    asf = a_s.reshape(1, _H * _C)
    adf = a_d.reshape(1, _H * _C)
    xh, al = _project(xp, W, asf, adf, eh)
    als = al[:, :_H]
    ald = al[:, _H:]
    alpha = jax.nn.leaky_relu(als[src] + ald[dst], 0.2)
    amax = jax.ops.segment_max(alpha, dst, num_segments=n)
    amax = jnp.where(jnp.isfinite(amax), amax, 0.0)
    ex = jnp.exp(alpha - amax[dst])
    den = jax.ops.segment_sum(ex, dst, num_segments=n)
    attn = ex / (den[dst] + 1e-16)
    xh3 = xh[:n].reshape(n, _H, _C)
    out = jax.ops.segment_sum(xh3[src] * attn[:, :, None], dst, num_segments=n)
    return out.mean(axis=1) + b


def _bn_relu(h, g, b):
    h = jax.nn.relu(h)
    mu = jnp.mean(h, axis=0)
    var = jnp.var(h, axis=0)
    return (h - mu) / jnp.sqrt(var + 1e-5) * g + b


def kernel(x, edge_index, batch, W1, as1, ad1, b1, g1, be1, W2, as2, ad2, b2, g2, be2, W3, as3, ad3, b3, g3, be3, lnW, lnb, l0W, l0b, l1W, l1b):
    n = x.shape[0]
    npad = ((n + _BLK - 1) // _BLK) * _BLK
    loop = jnp.arange(n, dtype=edge_index.dtype)
    src = jnp.concatenate([edge_index[0], loop])
    dst = jnp.concatenate([edge_index[1], loop])

    eh = (jnp.arange(_H * _C, dtype=jnp.int32)[:, None] // _C
          == jnp.arange(_H, dtype=jnp.int32)[None, :]).astype(jnp.float32)

    xp = jnp.pad(x, ((0, npad - n), (0, 0)))
    h = _bn_relu(_gat_layer(xp, n, src, dst, W1, as1, ad1, b1, eh), g1, be1)
    hp = jnp.pad(h, ((0, npad - n), (0, 0)))
    h = _bn_relu(_gat_layer(hp, n, src, dst, W2, as2, ad2, b2, eh), g2, be2)
    hp = jnp.pad(h, ((0, npad - n), (0, 0)))
    h = _bn_relu(_gat_layer(hp, n, src, dst, W3, as3, ad3, b3, eh), g3, be3)

    ng = 64
    gmax = jax.ops.segment_max(h, batch, num_segments=ng)
    gmax = jnp.where(jnp.isfinite(gmax), gmax, 0.0)
    cnt = jax.ops.segment_sum(jnp.ones((n,), jnp.float32), batch, num_segments=ng)
    gmean = jax.ops.segment_sum(h, batch, num_segments=ng) / jnp.maximum(cnt, 1.0)[:, None]
    hg = jnp.concatenate([gmax, gmean], axis=-1)
    hg = jax.nn.relu(hg @ l0W + l0b)
    root = jax.ops.segment_min(jnp.arange(n, dtype=batch.dtype), batch, num_segments=ng)
    news = jax.nn.relu(x[root] @ lnW + lnb)
    out = jnp.concatenate([hg, news], axis=-1) @ l1W + l1b
    return jax.nn.sigmoid(out)
